# baseline (speedup 1.0000x reference)
import functools

import jax
import jax.numpy as jnp
from jax import lax
from jax.experimental import pallas as pl
from jax.experimental.pallas import tpu as pltpu
from jax.experimental.pallas import tpu_sc as plsc

BATCH = 16384
OUT_DIM = 64

_NUM_CORES = 2
_NUM_SUBCORES = 16
_NUM_WORKERS = _NUM_CORES * _NUM_SUBCORES
_B_PER_W = BATCH // _NUM_WORKERS
_IDX_CHUNK = 128
_N_CHUNKS = _B_PER_W // _IDX_CHUNK

_mesh = plsc.VectorSubcoreMesh(core_axis_name="c", subcore_axis_name="s")


@functools.partial(
    pl.kernel,
    out_type=jax.ShapeDtypeStruct((BATCH, OUT_DIM), jnp.float32),
    mesh=_mesh,
    compiler_params=pltpu.CompilerParams(use_tc_tiling_on_sc=False),
    scratch_types=[
        pltpu.VMEM((_N_CHUNKS, _IDX_CHUNK), jnp.int32),
        pltpu.VMEM((_B_PER_W, OUT_DIM), jnp.float32),
        pltpu.SemaphoreType.DMA,
    ],
)
def _embed_gather(labels_hbm, table_hbm, out_hbm, idx_v, rows_v, sem):
    wid = lax.axis_index("s") * _NUM_CORES + lax.axis_index("c")
    base = wid * _B_PER_W
    pltpu.sync_copy(labels_hbm.at[pl.ds(wid * _N_CHUNKS, _N_CHUNKS)], idx_v)
    copies = []
    for j in range(_N_CHUNKS):
        copies.append(
            pltpu.async_copy(
                table_hbm.at[idx_v.at[j]],
                rows_v.at[pl.ds(j * _IDX_CHUNK, _IDX_CHUNK)],
                sem,
            )
        )
    for c in copies:
        c.wait()
    pltpu.sync_copy(rows_v, out_hbm.at[pl.ds(base, _B_PER_W)])


def kernel(labels, table, training=0):
    del training
    labels2d = labels.astype(jnp.int32).reshape(
        _NUM_WORKERS * _N_CHUNKS, _IDX_CHUNK
    )
    return _embed_gather(labels2d, table)


# speedup vs baseline: 2.0469x; 2.0469x over previous
import functools

import jax
import jax.numpy as jnp
from jax import lax
from jax.experimental import pallas as pl
from jax.experimental.pallas import tpu as pltpu
from jax.experimental.pallas import tpu_sc as plsc

BATCH = 16384
OUT_DIM = 64

_NUM_CORES = 2
_NUM_SUBCORES = 16
_NUM_WORKERS = _NUM_CORES * _NUM_SUBCORES
_B_PER_W = BATCH // _NUM_WORKERS
_CHUNK = 64
_N_CHUNKS = _B_PER_W // _CHUNK
_LANES = 16
_SLAB = 8

_mesh = plsc.VectorSubcoreMesh(core_axis_name="c", subcore_axis_name="s")


@functools.partial(
    pl.kernel,
    out_type=jax.ShapeDtypeStruct((BATCH // _SLAB, _SLAB, OUT_DIM), jnp.float32),
    mesh=_mesh,
    compiler_params=pltpu.CompilerParams(needs_layout_passes=False),
    scratch_types=[
        pltpu.VMEM((_B_PER_W,), jnp.int32),
        pltpu.VMEM((_CHUNK, _SLAB, OUT_DIM), jnp.float32),
        pltpu.VMEM((_CHUNK // _SLAB, _SLAB, OUT_DIM), jnp.float32),
        pltpu.SemaphoreType.DMA,
    ],
)
def _embed_gather(labels_hbm, table_hbm, out_hbm, lab_v, slab_v,
                  out_v, sem):
    wid = lax.axis_index("s") * _NUM_CORES + lax.axis_index("c")
    base = wid * _B_PER_W
    pltpu.sync_copy(labels_hbm.at[pl.ds(base, _B_PER_W)], lab_v)

    def chunk_body(k, carry):
        copies = []
        for g in range(_CHUNK // _LANES):
            tv = lab_v[pl.ds(k * _CHUNK + g * _LANES, _LANES)] >> 3
            for i in range(_LANES):
                copies.append(
                    pltpu.async_copy(
                        table_hbm.at[tv[i]], slab_v.at[g * _LANES + i], sem
                    )
                )
        for c in copies:
            c.wait()
        for g in range(_CHUNK // _LANES):
            lv = lab_v[pl.ds(k * _CHUNK + g * _LANES, _LANES)]
            r_vec = lv & 7
            i_vec = lax.iota(jnp.int32, _LANES) + g * _LANES
            os_vec = i_vec >> 3
            or_vec = i_vec & 7
            for c in range(OUT_DIM):
                c_vec = jnp.full((_LANES,), c, jnp.int32)
                x = plsc.load_gather(slab_v, [i_vec, r_vec, c_vec])
                plsc.store_scatter(out_v, [os_vec, or_vec, c_vec], x)
        pltpu.sync_copy(
            out_v,
            out_hbm.at[pl.ds(wid * (_B_PER_W // _SLAB) + k * (_CHUNK // _SLAB),
                             _CHUNK // _SLAB)],
        )
        return carry

    lax.fori_loop(0, _N_CHUNKS, chunk_body, 0)


def kernel(labels, table, training=0):
    del training
    table3 = table.reshape(table.shape[0] // _SLAB, _SLAB, OUT_DIM)
    out3 = _embed_gather(labels.astype(jnp.int32), table3)
    return out3.reshape(BATCH, OUT_DIM)
